# baseline (speedup 1.0000x reference)
import functools

import jax
import jax.numpy as jnp
from jax import lax
from jax.experimental import pallas as pl
from jax.experimental.pallas import tpu as pltpu
from jax.experimental.pallas import tpu_sc as plsc

N1 = 10000
N2 = 10000
D = 128
F3 = 32
TN = 16
BINS = 16

_BM = 1000
_BN = 2048
_NPAD = 10240


def _hist_body(a1_ref, a2_ref, hist_ref, mm_sm, acc_sm, *, n_valid, gi, gj):
    p = pl.program_id(0)
    i = pl.program_id(1)
    j = pl.program_id(2)
    first = (i == 0) & (j == 0)
    last = (i == gi - 1) & (j == gj - 1)

    s = jnp.dot(a1_ref[...], a2_ref[...], preferred_element_type=jnp.float32)
    col = j * _BN + jax.lax.broadcasted_iota(jnp.int32, (_BM, _BN), 1)
    valid = col < n_valid

    @pl.when(p == 0)
    def _minmax():
        @pl.when(first)
        def _init():
            mm_sm[0] = jnp.inf
            mm_sm[1] = -jnp.inf

        bmin = jnp.min(jnp.where(valid, s, jnp.inf))
        bmax = jnp.max(jnp.where(valid, s, -jnp.inf))
        mm_sm[0] = jnp.minimum(mm_sm[0], bmin)
        mm_sm[1] = jnp.maximum(mm_sm[1], bmax)

    @pl.when(p == 1)
    def _bin():
        lo = mm_sm[0]
        hi = mm_sm[1]
        scale = BINS / jnp.maximum(hi - lo, 1e-30)
        idx = jnp.floor((s - lo) * scale).astype(jnp.int32)
        idx = jnp.clip(idx, 0, BINS - 1)
        idx = jnp.where(valid, idx, -1)

        @pl.when(first)
        def _init():
            for b in range(BINS):
                acc_sm[b] = 0

        for b in range(BINS):
            acc_sm[b] = acc_sm[b] + jnp.sum((idx == b).astype(jnp.int32))

        @pl.when(last)
        def _write():
            for b in range(BINS):
                hist_ref[0, b] = acc_sm[b]


def _fused_histogram(a1, a2):
    m, k = a1.shape
    n = a2.shape[0]
    a2t = jnp.zeros((k, _NPAD), a1.dtype).at[:, :n].set(a2.T)
    gi, gj = m // _BM, _NPAD // _BN
    hist = pl.pallas_call(
        functools.partial(_hist_body, n_valid=n, gi=gi, gj=gj),
        grid=(2, gi, gj),
        in_specs=[
            pl.BlockSpec((_BM, k), lambda p, i, j: (i, 0)),
            pl.BlockSpec((k, _BN), lambda p, i, j: (0, j)),
        ],
        out_specs=pl.BlockSpec(memory_space=pltpu.SMEM),
        out_shape=jax.ShapeDtypeStruct((1, BINS), jnp.int32),
        scratch_shapes=[
            pltpu.SMEM((2,), jnp.float32),
            pltpu.SMEM((BINS,), jnp.int32),
        ],
    )(a1, a2t)
    hist = jnp.minimum(hist, 2**24).astype(jnp.float32)
    return hist / jnp.sum(hist)


_NTILE = 16
_EPT = 320000 // _NTILE
_CH = 128
_NCH = 160
_ACC_ROWS = 10112
_ZROWS = 8
_NPADR = 16


def _prop_body(g_hbm, src_hbm, dst_hbm, out_hbm,
               src_v, dst_v, gbuf0, gbuf1, zbuf, acc, sem0, sem1, *, F, Q):
    CPQ = _NCH // Q
    c = lax.axis_index("c")
    s = lax.axis_index("s")
    wid = c * _NTILE + s

    z16 = jnp.zeros((16,), jnp.float32)
    for r in range(_ZROWS):
        for l in range(F // 16):
            zbuf[r, pl.ds(l * 16, 16)] = z16
    zbase = s * 632

    def zero_step(i, _):
        pltpu.sync_copy(zbuf, acc.at[pl.ds(zbase + i * _ZROWS, _ZROWS)])
        return 0

    lax.fori_loop(0, 632 // _ZROWS, zero_step, 0)
    plsc.subcore_barrier()

    def start_gather(j, buf, sem):
        pltpu.make_async_copy(
            g_hbm.at[src_v.at[pl.ds(j * _CH, _CH)]], buf, sem).start()

    def wait_gather(j, buf, sem):
        pltpu.make_async_copy(
            g_hbm.at[src_v.at[pl.ds(j * _CH, _CH)]], buf, sem).wait()

    def scatter_add(j, buf):
        pltpu.sync_copy(buf, acc.at[dst_v.at[j]], add=True)

    for q in range(Q):
        pltpu.sync_copy(src_hbm.at[wid, pl.ds(q * CPQ * _CH, CPQ * _CH)], src_v)
        pltpu.sync_copy(dst_hbm.at[wid, pl.ds(q * CPQ, CPQ)], dst_v)
        start_gather(0, gbuf0, sem0)

        def pair(i, _):
            j0 = 2 * i
            start_gather(j0 + 1, gbuf1, sem1)
            wait_gather(j0, gbuf0, sem0)
            scatter_add(j0, gbuf0)

            @pl.when(j0 + 2 < CPQ)
            def _():
                start_gather(j0 + 2, gbuf0, sem0)

            wait_gather(j0 + 1, gbuf1, sem1)
            scatter_add(j0 + 1, gbuf1)
            return 0

        lax.fori_loop(0, CPQ // 2, pair, 0)

    plsc.subcore_barrier()

    obase = s * 624
    for k, rows in ((0, 128), (128, 128), (256, 128), (384, 128), (512, 112)):
        pltpu.sync_copy(acc.at[pl.ds(obase + k, rows)], gbuf0.at[pl.ds(0, rows)])
        pltpu.sync_copy(gbuf0.at[pl.ds(0, rows)],
                        out_hbm.at[pl.ds(c * 10000 + obase + k, rows)])

    @pl.when(s == 0)
    def _tail():
        pltpu.sync_copy(acc.at[pl.ds(9984, 16)], gbuf1.at[pl.ds(0, 16)])
        pltpu.sync_copy(gbuf1.at[pl.ds(0, 16)],
                        out_hbm.at[pl.ds(c * 10000 + 9984, 16)])

    @pl.when((s == 0) & (c == 0))
    def _padrows():
        pltpu.sync_copy(zbuf, out_hbm.at[pl.ds(20000, _ZROWS)])
        pltpu.sync_copy(zbuf, out_hbm.at[pl.ds(20008, _ZROWS)])


@functools.partial(jax.jit, static_argnums=(3,))
def _sc_propagate(g_stack, src_hbm, dst_hbm, F):
    Q = 4 if F > 64 else 1
    CPQ = _NCH // Q
    mesh = plsc.VectorSubcoreMesh(core_axis_name="c", subcore_axis_name="s",
                                  num_cores=2, num_subcores=_NTILE)
    return pl.kernel(
        functools.partial(_prop_body, F=F, Q=Q),
        out_type=jax.ShapeDtypeStruct((2 * 10000 + _NPADR, F), jnp.float32),
        mesh=mesh,
        compiler_params=pltpu.CompilerParams(use_tc_tiling_on_sc=False),
        scratch_types=[
            pltpu.VMEM((CPQ * _CH,), jnp.int32),
            pltpu.VMEM((CPQ, _CH), jnp.int32),
            pltpu.VMEM((_CH, F), jnp.float32),
            pltpu.VMEM((_CH, F), jnp.float32),
            pltpu.VMEM((_ZROWS, F), jnp.float32),
            pltpu.VMEM_SHARED((_ACC_ROWS, F), jnp.float32),
            pltpu.SemaphoreType.DMA,
            pltpu.SemaphoreType.DMA,
        ],
    )(g_stack, src_hbm, dst_hbm)


def _prep_edges(ei1, ei2):
    pad = _NCH * _CH - _EPT

    def per_graph(ei, src_off):
        src = ei[0].astype(jnp.int32).reshape(_NTILE, _EPT) + src_off
        dst = ei[1].astype(jnp.int32).reshape(_NTILE, _EPT)
        src = jnp.pad(src, ((0, 0), (0, pad)), constant_values=2 * 10000)
        dst = jnp.pad(dst, ((0, 0), (0, pad)), constant_values=10000)
        return src, dst

    s1, d1 = per_graph(ei1, 0)
    s2, d2 = per_graph(ei2, 10000)
    src = jnp.concatenate([s1, s2], 0)
    dst = jnp.concatenate([d1, d2], 0).reshape(32, _NCH, _CH)
    return src, dst


def _attention(emb, att_W):
    gc = jnp.mean(emb @ att_W, axis=0)
    tg = jnp.tanh(gc)
    sig = jax.nn.sigmoid(emb @ tg[:, None])
    return emb.T @ sig


def _ntn(e1, e2, p):
    scoring = (e1.T @ p["ntn_W"].reshape(F3, F3 * TN)).reshape(F3, TN)
    scoring = scoring.T @ e2
    combined = jnp.concatenate([e1, e2], axis=0)
    block = p["ntn_V"] @ combined
    return jax.nn.relu(scoring + block + p["ntn_b"])


def _sc_conv_passes(x1, x2, ei1, ei2, p):
    src_hbm, dst_hbm = _prep_edges(ei1, ei2)
    deg1 = jnp.zeros((10000,), jnp.float32).at[ei1[1]].add(1.0) + 1.0
    deg2 = jnp.zeros((10000,), jnp.float32).at[ei2[1]].add(1.0) + 1.0
    dinv = jnp.concatenate(
        [jax.lax.rsqrt(deg1), jax.lax.rsqrt(deg2),
         jnp.zeros((_NPADR,), jnp.float32)])[:, None]
    h = jnp.concatenate([x1, x2, jnp.zeros((_NPADR, x1.shape[1]), jnp.float32)], 0)

    for (W, b), act in (((p["W1"], p["b1"]), True),
                        ((p["W2"], p["b2"]), True),
                        ((p["W3"], p["b3"]), False)):
        F = W.shape[1]
        g = (h @ W) * dinv
        acc = _sc_propagate(g, src_hbm, dst_hbm, F)
        o = dinv * (acc + g) + b
        h = jax.nn.relu(o) if act else o
    return h[:10000], h[10000:20000]


def kernel(features_1, features_2, edge_index_1, edge_index_2, avg_v, params):
    p = params
    a1, a2 = _sc_conv_passes(features_1, features_2, edge_index_1, edge_index_2, p)
    p1 = _attention(a1, p["att_W"])
    p2 = _attention(a2, p["att_W"])
    scores = _ntn(p1, p2, p).T
    hist = _fused_histogram(a1, a2)
    scores = jnp.concatenate([scores, hist], axis=1).reshape(1, -1)
    s = jax.nn.relu(scores @ p["fc1_W"] + p["fc1_b"])
    s = jax.nn.relu(s @ p["fc2_W"] + p["fc2_b"])
    s = jax.nn.relu(s @ p["fc3_W"] + p["fc3_b"])
    score = jax.nn.sigmoid((s @ p["sc_W"] + p["sc_b"]).reshape(-1))
    pre_ged = -jnp.log(score) * avg_v
    return score, pre_ged


# speedup vs baseline: 1.0679x; 1.0679x over previous
import functools

import jax
import jax.numpy as jnp
from jax import lax
from jax.experimental import pallas as pl
from jax.experimental.pallas import tpu as pltpu
from jax.experimental.pallas import tpu_sc as plsc

N1 = 10000
N2 = 10000
D = 128
F3 = 32
TN = 16
BINS = 16

_BM = 1000
_BN = 2048
_NPAD = 10240


def _hist_body(a1_ref, a2_ref, hist_ref, mm_sm, acc_sm, *, n_valid, gi, gj):
    p = pl.program_id(0)
    i = pl.program_id(1)
    j = pl.program_id(2)
    first = (i == 0) & (j == 0)
    last = (i == gi - 1) & (j == gj - 1)

    s = jnp.dot(a1_ref[...], a2_ref[...], preferred_element_type=jnp.float32)
    col = j * _BN + jax.lax.broadcasted_iota(jnp.int32, (_BM, _BN), 1)
    valid = col < n_valid

    @pl.when(p == 0)
    def _minmax():
        @pl.when(first)
        def _init():
            mm_sm[0] = jnp.inf
            mm_sm[1] = -jnp.inf

        bmin = jnp.min(jnp.where(valid, s, jnp.inf))
        bmax = jnp.max(jnp.where(valid, s, -jnp.inf))
        mm_sm[0] = jnp.minimum(mm_sm[0], bmin)
        mm_sm[1] = jnp.maximum(mm_sm[1], bmax)

    @pl.when(p == 1)
    def _bin():
        lo = mm_sm[0]
        hi = mm_sm[1]
        scale = BINS / jnp.maximum(hi - lo, 1e-30)
        idx = jnp.floor((s - lo) * scale).astype(jnp.int32)
        idx = jnp.clip(idx, 0, BINS - 1)
        idx = jnp.where(valid, idx, -1)

        @pl.when(first)
        def _init():
            for b in range(BINS):
                acc_sm[b] = 0

        for b in range(BINS):
            acc_sm[b] = acc_sm[b] + jnp.sum((idx == b).astype(jnp.int32))

        @pl.when(last)
        def _write():
            for b in range(BINS):
                hist_ref[0, b] = acc_sm[b]


def _fused_histogram(a1, a2):
    m, k = a1.shape
    n = a2.shape[0]
    a2t = jnp.zeros((k, _NPAD), a1.dtype).at[:, :n].set(a2.T)
    gi, gj = m // _BM, _NPAD // _BN
    hist = pl.pallas_call(
        functools.partial(_hist_body, n_valid=n, gi=gi, gj=gj),
        grid=(2, gi, gj),
        in_specs=[
            pl.BlockSpec((_BM, k), lambda p, i, j: (i, 0)),
            pl.BlockSpec((k, _BN), lambda p, i, j: (0, j)),
        ],
        out_specs=pl.BlockSpec(memory_space=pltpu.SMEM),
        out_shape=jax.ShapeDtypeStruct((1, BINS), jnp.int32),
        scratch_shapes=[
            pltpu.SMEM((2,), jnp.float32),
            pltpu.SMEM((BINS,), jnp.int32),
        ],
    )(a1, a2t)
    hist = jnp.minimum(hist, 2**24).astype(jnp.float32)
    return hist / jnp.sum(hist)


_NTILE = 16
_EPT = 320000 // _NTILE
_CH = 128
_NCH = 158
_NCHA = 160
_ACC_ROWS = 10112
_ZROWS = 8
_NPADR = 16


def _prop_body(g_hbm, src_hbm, dst_hbm, out_hbm,
               src_v, dst_v, gbuf0, gbuf1, zbuf, acc, sem0, sem1, *, F):
    c = lax.axis_index("c")
    s = lax.axis_index("s")
    wid = c * _NTILE + s

    pltpu.sync_copy(src_hbm.at[wid], src_v)
    pltpu.sync_copy(dst_hbm.at[wid], dst_v)

    z16 = jnp.zeros((16,), jnp.float32)
    for r in range(_ZROWS):
        for l in range(F // 16):
            zbuf[r, pl.ds(l * 16, 16)] = z16
    zbase = s * 632

    def zero_step(i, _):
        pltpu.sync_copy(zbuf, acc.at[pl.ds(zbase + i * _ZROWS, _ZROWS)])
        return 0

    lax.fori_loop(0, 632 // _ZROWS, zero_step, 0)
    plsc.subcore_barrier()

    def start_gather(j, buf, sem):
        pltpu.make_async_copy(
            g_hbm.at[src_v.at[pl.ds(j * _CH, _CH)]], buf, sem).start()

    def wait_gather(j, buf, sem):
        pltpu.make_async_copy(
            g_hbm.at[src_v.at[pl.ds(j * _CH, _CH)]], buf, sem).wait()

    def scatter_add(j, buf):
        pltpu.sync_copy(buf, acc.at[dst_v.at[j]], add=True)

    start_gather(0, gbuf0, sem0)

    def pair(i, _):
        j0 = 2 * i
        start_gather(j0 + 1, gbuf1, sem1)
        wait_gather(j0, gbuf0, sem0)
        scatter_add(j0, gbuf0)
        start_gather(j0 + 2, gbuf0, sem0)
        wait_gather(j0 + 1, gbuf1, sem1)
        scatter_add(j0 + 1, gbuf1)
        return 0

    lax.fori_loop(0, _NCH // 2, pair, 0)
    wait_gather(_NCH, gbuf0, sem0)

    plsc.subcore_barrier()

    obase = s * 624
    for k, rows in ((0, 128), (128, 128), (256, 128), (384, 128), (512, 112)):
        pltpu.sync_copy(acc.at[pl.ds(obase + k, rows)], gbuf0.at[pl.ds(0, rows)])
        pltpu.sync_copy(gbuf0.at[pl.ds(0, rows)],
                        out_hbm.at[pl.ds(c * 10000 + obase + k, rows)])

    @pl.when(s == 0)
    def _tail():
        pltpu.sync_copy(acc.at[pl.ds(9984, 16)], gbuf1.at[pl.ds(0, 16)])
        pltpu.sync_copy(gbuf1.at[pl.ds(0, 16)],
                        out_hbm.at[pl.ds(c * 10000 + 9984, 16)])

    @pl.when((s == 0) & (c == 0))
    def _padrows():
        pltpu.sync_copy(zbuf, out_hbm.at[pl.ds(20000, _ZROWS)])
        pltpu.sync_copy(zbuf, out_hbm.at[pl.ds(20008, _ZROWS)])


@functools.partial(jax.jit, static_argnums=(3,))
def _sc_propagate(g_stack, src_hbm, dst_hbm, F):
    mesh = plsc.VectorSubcoreMesh(core_axis_name="c", subcore_axis_name="s",
                                  num_cores=2, num_subcores=_NTILE)
    return pl.kernel(
        functools.partial(_prop_body, F=F),
        out_type=jax.ShapeDtypeStruct((2 * 10000 + _NPADR, F), jnp.float32),
        mesh=mesh,
        compiler_params=pltpu.CompilerParams(use_tc_tiling_on_sc=False),
        scratch_types=[
            pltpu.VMEM((_NCHA * _CH,), jnp.int32),
            pltpu.VMEM((_NCHA, _CH), jnp.int32),
            pltpu.VMEM((_CH, F), jnp.float32),
            pltpu.VMEM((_CH, F), jnp.float32),
            pltpu.VMEM((_ZROWS, F), jnp.float32),
            pltpu.VMEM_SHARED((_ACC_ROWS, F), jnp.float32),
            pltpu.SemaphoreType.DMA,
            pltpu.SemaphoreType.DMA,
        ],
    )(g_stack, src_hbm, dst_hbm)


def _prep_edges(ei1, ei2):
    pad = _NCHA * _CH - _EPT

    def per_graph(ei, src_off):
        src = ei[0].astype(jnp.int32).reshape(_NTILE, _EPT) + src_off
        dst = ei[1].astype(jnp.int32).reshape(_NTILE, _EPT)
        src = jnp.pad(src, ((0, 0), (0, pad)), constant_values=2 * 10000)
        dst = jnp.pad(dst, ((0, 0), (0, pad)), constant_values=10000)
        return src, dst

    s1, d1 = per_graph(ei1, 0)
    s2, d2 = per_graph(ei2, 10000)
    src = jnp.concatenate([s1, s2], 0)
    dst = jnp.concatenate([d1, d2], 0).reshape(32, _NCHA, _CH)
    return src, dst


def _attention(emb, att_W):
    gc = jnp.mean(emb @ att_W, axis=0)
    tg = jnp.tanh(gc)
    sig = jax.nn.sigmoid(emb @ tg[:, None])
    return emb.T @ sig


def _ntn(e1, e2, p):
    scoring = (e1.T @ p["ntn_W"].reshape(F3, F3 * TN)).reshape(F3, TN)
    scoring = scoring.T @ e2
    combined = jnp.concatenate([e1, e2], axis=0)
    block = p["ntn_V"] @ combined
    return jax.nn.relu(scoring + block + p["ntn_b"])


def _sc_conv_passes(x1, x2, ei1, ei2, p):
    src_hbm, dst_hbm = _prep_edges(ei1, ei2)
    deg1 = jnp.zeros((10000,), jnp.float32).at[ei1[1]].add(1.0) + 1.0
    deg2 = jnp.zeros((10000,), jnp.float32).at[ei2[1]].add(1.0) + 1.0
    dinv = jnp.concatenate(
        [jax.lax.rsqrt(deg1), jax.lax.rsqrt(deg2),
         jnp.zeros((_NPADR,), jnp.float32)])[:, None]
    h = jnp.concatenate([x1, x2, jnp.zeros((_NPADR, x1.shape[1]), jnp.float32)], 0)

    for (W, b), act in (((p["W1"], p["b1"]), True),
                        ((p["W2"], p["b2"]), True),
                        ((p["W3"], p["b3"]), False)):
        F = W.shape[1]
        parts = []
        for lo in range(0, F, 64):
            Fc = min(64, F - lo)
            g = (h @ W[:, lo:lo + Fc]) * dinv
            acc = _sc_propagate(g, src_hbm, dst_hbm, Fc)
            parts.append(dinv * (acc + g) + b[lo:lo + Fc])
        o = jnp.concatenate(parts, axis=1) if len(parts) > 1 else parts[0]
        h = jax.nn.relu(o) if act else o
    return h[:10000], h[10000:20000]


def kernel(features_1, features_2, edge_index_1, edge_index_2, avg_v, params):
    p = params
    a1, a2 = _sc_conv_passes(features_1, features_2, edge_index_1, edge_index_2, p)
    p1 = _attention(a1, p["att_W"])
    p2 = _attention(a2, p["att_W"])
    scores = _ntn(p1, p2, p).T
    hist = _fused_histogram(a1, a2)
    scores = jnp.concatenate([scores, hist], axis=1).reshape(1, -1)
    s = jax.nn.relu(scores @ p["fc1_W"] + p["fc1_b"])
    s = jax.nn.relu(s @ p["fc2_W"] + p["fc2_b"])
    s = jax.nn.relu(s @ p["fc3_W"] + p["fc3_b"])
    score = jax.nn.sigmoid((s @ p["sc_W"] + p["sc_b"]).reshape(-1))
    pre_ged = -jnp.log(score) * avg_v
    return score, pre_ged


# speedup vs baseline: 1.2963x; 1.2139x over previous
import functools

import jax
import jax.numpy as jnp
from jax import lax
from jax.experimental import pallas as pl
from jax.experimental.pallas import tpu as pltpu
from jax.experimental.pallas import tpu_sc as plsc

N1 = 10000
N2 = 10000
D = 128
F3 = 32
TN = 16
BINS = 16

_BM = 1000
_BN = 2048
_NPAD = 10240


_SLAB = 8


def _hist_body(a1_ref, a2_ref, hist_ref, mm_sm, acc_sm, sbuf, pk, *, n_valid,
               gi, gj):
    p = pl.program_id(0)
    i = pl.program_id(1)
    j = pl.program_id(2)
    first = (i == 0) & (j == 0)
    last = (i == gi - 1) & (j == gj - 1)

    col = j * _BN + jax.lax.broadcasted_iota(jnp.int32, (_SLAB, _BN), 1)
    valid = col < n_valid

    @pl.when(p == 0)
    def _minmax():
        @pl.when(first)
        def _init():
            mm_sm[0] = jnp.inf
            mm_sm[1] = -jnp.inf

        s = jnp.dot(a1_ref[...], a2_ref[...],
                    preferred_element_type=jnp.float32)
        colf = j * _BN + jax.lax.broadcasted_iota(jnp.int32, (_BM, _BN), 1)
        vf = colf < n_valid
        bmin = jnp.min(jnp.where(vf, s, jnp.inf))
        bmax = jnp.max(jnp.where(vf, s, -jnp.inf))
        mm_sm[0] = jnp.minimum(mm_sm[0], bmin)
        mm_sm[1] = jnp.maximum(mm_sm[1], bmax)

    @pl.when(p == 1)
    def _bin():
        lo = mm_sm[0]
        hi = mm_sm[1]
        scale = BINS / jnp.maximum(hi - lo, 1e-30)

        sbuf[...] = jnp.dot(a1_ref[...], a2_ref[...],
                            preferred_element_type=jnp.float32)

        @pl.when(first)
        def _init():
            for b in range(BINS):
                acc_sm[b] = 0

        zero = jnp.zeros((_SLAB, _BN), jnp.int32)
        for k in range(4):
            pk[k] = zero

        def slab(t, _):
            s = sbuf[pl.ds(t * _SLAB, _SLAB), :]
            idx = jnp.minimum(((s - lo) * scale).astype(jnp.int32), BINS - 1)
            one = jnp.where(valid, jnp.int32(1) << ((idx & 3) * 8), 0)
            q = idx >> 2
            for k in range(4):
                pk[k] = pk[k] + jnp.where(q == k, one, zero)
            return 0

        lax.fori_loop(0, _BM // _SLAB, slab, 0)

        for k in range(4):
            packed = pk[k]
            for f in range(4):
                cnt = (packed >> (8 * f)) & 0xFF
                b = 4 * k + f
                acc_sm[b] = acc_sm[b] + jnp.sum(cnt)

        @pl.when(last)
        def _write():
            for b in range(BINS):
                hist_ref[0, b] = acc_sm[b]


def _fused_histogram(a1, a2):
    m, k = a1.shape
    n = a2.shape[0]
    a2t = jnp.zeros((k, _NPAD), a1.dtype).at[:, :n].set(a2.T)
    gi, gj = m // _BM, _NPAD // _BN
    hist = pl.pallas_call(
        functools.partial(_hist_body, n_valid=n, gi=gi, gj=gj),
        grid=(2, gi, gj),
        in_specs=[
            pl.BlockSpec((_BM, k), lambda p, i, j: (i, 0)),
            pl.BlockSpec((k, _BN), lambda p, i, j: (0, j)),
        ],
        out_specs=pl.BlockSpec(memory_space=pltpu.SMEM),
        out_shape=jax.ShapeDtypeStruct((1, BINS), jnp.int32),
        scratch_shapes=[
            pltpu.SMEM((2,), jnp.float32),
            pltpu.SMEM((BINS,), jnp.int32),
            pltpu.VMEM((_BM, _BN), jnp.float32),
            pltpu.VMEM((4, _SLAB, _BN), jnp.int32),
        ],
    )(a1, a2t)
    hist = jnp.minimum(hist, 2**24).astype(jnp.float32)
    return hist / jnp.sum(hist)


_NTILE = 16
_EPT = 320000 // _NTILE
_CH = 128
_NCH = 158
_NCHA = 160
_ACC_ROWS = 10112
_ZROWS = 8
_NPADR = 16


def _prop_body(g_hbm, src_hbm, dst_hbm, out_hbm,
               src_v, dst_v, gbuf0, gbuf1, zbuf, acc, sem0, sem1, *, F):
    c = lax.axis_index("c")
    s = lax.axis_index("s")
    wid = c * _NTILE + s

    pltpu.sync_copy(src_hbm.at[wid], src_v)
    pltpu.sync_copy(dst_hbm.at[wid], dst_v)

    z16 = jnp.zeros((16,), jnp.float32)
    for r in range(_ZROWS):
        for l in range(F // 16):
            zbuf[r, pl.ds(l * 16, 16)] = z16
    zbase = s * 632

    def zero_step(i, _):
        pltpu.sync_copy(zbuf, acc.at[pl.ds(zbase + i * _ZROWS, _ZROWS)])
        return 0

    lax.fori_loop(0, 632 // _ZROWS, zero_step, 0)
    plsc.subcore_barrier()

    def start_gather(j, buf, sem):
        pltpu.make_async_copy(
            g_hbm.at[src_v.at[pl.ds(j * _CH, _CH)]], buf, sem).start()

    def wait_gather(j, buf, sem):
        pltpu.make_async_copy(
            g_hbm.at[src_v.at[pl.ds(j * _CH, _CH)]], buf, sem).wait()

    def scatter_add(j, buf):
        pltpu.sync_copy(buf, acc.at[dst_v.at[j]], add=True)

    start_gather(0, gbuf0, sem0)

    def pair(i, _):
        j0 = 2 * i
        start_gather(j0 + 1, gbuf1, sem1)
        wait_gather(j0, gbuf0, sem0)
        scatter_add(j0, gbuf0)
        start_gather(j0 + 2, gbuf0, sem0)
        wait_gather(j0 + 1, gbuf1, sem1)
        scatter_add(j0 + 1, gbuf1)
        return 0

    lax.fori_loop(0, _NCH // 2, pair, 0)
    wait_gather(_NCH, gbuf0, sem0)

    plsc.subcore_barrier()

    obase = s * 624
    for k, rows in ((0, 128), (128, 128), (256, 128), (384, 128), (512, 112)):
        pltpu.sync_copy(acc.at[pl.ds(obase + k, rows)], gbuf0.at[pl.ds(0, rows)])
        pltpu.sync_copy(gbuf0.at[pl.ds(0, rows)],
                        out_hbm.at[pl.ds(c * 10000 + obase + k, rows)])

    @pl.when(s == 0)
    def _tail():
        pltpu.sync_copy(acc.at[pl.ds(9984, 16)], gbuf1.at[pl.ds(0, 16)])
        pltpu.sync_copy(gbuf1.at[pl.ds(0, 16)],
                        out_hbm.at[pl.ds(c * 10000 + 9984, 16)])

    @pl.when((s == 0) & (c == 0))
    def _padrows():
        pltpu.sync_copy(zbuf, out_hbm.at[pl.ds(20000, _ZROWS)])
        pltpu.sync_copy(zbuf, out_hbm.at[pl.ds(20008, _ZROWS)])


@functools.partial(jax.jit, static_argnums=(3,))
def _sc_propagate(g_stack, src_hbm, dst_hbm, F):
    mesh = plsc.VectorSubcoreMesh(core_axis_name="c", subcore_axis_name="s",
                                  num_cores=2, num_subcores=_NTILE)
    return pl.kernel(
        functools.partial(_prop_body, F=F),
        out_type=jax.ShapeDtypeStruct((2 * 10000 + _NPADR, F), jnp.float32),
        mesh=mesh,
        compiler_params=pltpu.CompilerParams(use_tc_tiling_on_sc=False),
        scratch_types=[
            pltpu.VMEM((_NCHA * _CH,), jnp.int32),
            pltpu.VMEM((_NCHA, _CH), jnp.int32),
            pltpu.VMEM((_CH, F), jnp.float32),
            pltpu.VMEM((_CH, F), jnp.float32),
            pltpu.VMEM((_ZROWS, F), jnp.float32),
            pltpu.VMEM_SHARED((_ACC_ROWS, F), jnp.float32),
            pltpu.SemaphoreType.DMA,
            pltpu.SemaphoreType.DMA,
        ],
    )(g_stack, src_hbm, dst_hbm)


def _prep_edges(ei1, ei2):
    pad = _NCHA * _CH - _EPT

    def per_graph(ei, src_off):
        src = ei[0].astype(jnp.int32).reshape(_NTILE, _EPT) + src_off
        dst = ei[1].astype(jnp.int32).reshape(_NTILE, _EPT)
        src = jnp.pad(src, ((0, 0), (0, pad)), constant_values=2 * 10000)
        dst = jnp.pad(dst, ((0, 0), (0, pad)), constant_values=10000)
        return src, dst

    s1, d1 = per_graph(ei1, 0)
    s2, d2 = per_graph(ei2, 10000)
    src = jnp.concatenate([s1, s2], 0)
    dst = jnp.concatenate([d1, d2], 0).reshape(32, _NCHA, _CH)
    return src, dst


def _attention(emb, att_W):
    gc = jnp.mean(emb @ att_W, axis=0)
    tg = jnp.tanh(gc)
    sig = jax.nn.sigmoid(emb @ tg[:, None])
    return emb.T @ sig


def _ntn(e1, e2, p):
    scoring = (e1.T @ p["ntn_W"].reshape(F3, F3 * TN)).reshape(F3, TN)
    scoring = scoring.T @ e2
    combined = jnp.concatenate([e1, e2], axis=0)
    block = p["ntn_V"] @ combined
    return jax.nn.relu(scoring + block + p["ntn_b"])


def _sc_conv_passes(x1, x2, ei1, ei2, p):
    src_hbm, dst_hbm = _prep_edges(ei1, ei2)
    deg1 = jnp.zeros((10000,), jnp.float32).at[ei1[1]].add(1.0) + 1.0
    deg2 = jnp.zeros((10000,), jnp.float32).at[ei2[1]].add(1.0) + 1.0
    dinv = jnp.concatenate(
        [jax.lax.rsqrt(deg1), jax.lax.rsqrt(deg2),
         jnp.zeros((_NPADR,), jnp.float32)])[:, None]
    h = jnp.concatenate([x1, x2, jnp.zeros((_NPADR, x1.shape[1]), jnp.float32)], 0)

    for (W, b), act in (((p["W1"], p["b1"]), True),
                        ((p["W2"], p["b2"]), True),
                        ((p["W3"], p["b3"]), False)):
        F = W.shape[1]
        parts = []
        for lo in range(0, F, 64):
            Fc = min(64, F - lo)
            g = (h @ W[:, lo:lo + Fc]) * dinv
            acc = _sc_propagate(g, src_hbm, dst_hbm, Fc)
            parts.append(dinv * (acc + g) + b[lo:lo + Fc])
        o = jnp.concatenate(parts, axis=1) if len(parts) > 1 else parts[0]
        h = jax.nn.relu(o) if act else o
    return h[:10000], h[10000:20000]


def kernel(features_1, features_2, edge_index_1, edge_index_2, avg_v, params):
    p = params
    a1, a2 = _sc_conv_passes(features_1, features_2, edge_index_1, edge_index_2, p)
    p1 = _attention(a1, p["att_W"])
    p2 = _attention(a2, p["att_W"])
    scores = _ntn(p1, p2, p).T
    hist = _fused_histogram(a1, a2)
    scores = jnp.concatenate([scores, hist], axis=1).reshape(1, -1)
    s = jax.nn.relu(scores @ p["fc1_W"] + p["fc1_b"])
    s = jax.nn.relu(s @ p["fc2_W"] + p["fc2_b"])
    s = jax.nn.relu(s @ p["fc3_W"] + p["fc3_b"])
    score = jax.nn.sigmoid((s @ p["sc_W"] + p["sc_b"]).reshape(-1))
    pre_ged = -jnp.log(score) * avg_v
    return score, pre_ged
